# baseline (speedup 1.0000x reference)
import functools

import jax
import jax.numpy as jnp
from jax import lax
from jax.experimental import pallas as pl
from jax.experimental.pallas import tpu as pltpu
from jax.experimental.pallas import tpu_sc as plsc

SEQ = 16384
DIM = 32
NC = 2
NS = 16
NW = NC * NS
TPW = SEQ // NW
CHUNK = 128
NCHUNK = TPW // CHUNK
NBLK = TPW // 16


def _sc_body(seq_hbm, emb_hbm, sgn_hbm, out_hbm,
             idx_v, rows_v, sgn_v, part_v, sem, ssem):
    wid = lax.axis_index("s") * NC + lax.axis_index("c")
    base = wid * TPW

    for j in range(NCHUNK):
        pltpu.sync_copy(seq_hbm.at[pl.ds(base + j * CHUNK, CHUNK)], idx_v.at[j])

    sgn_copies = [
        pltpu.make_async_copy(sgn_hbm.at[idx_v.at[j]],
                              sgn_v.at[pl.ds(j * CHUNK, CHUNK)], ssem)
        for j in range(NCHUNK)
    ]
    for c in sgn_copies:
        c.start()

    def fire_block(i):
        j = i // (CHUNK // 16)
        q = i % (CHUNK // 16)
        c_vec = idx_v[j, pl.ds(q * 16, 16)]
        t0 = i * 16
        for k in range(16):
            pltpu.make_async_copy(emb_hbm.at[c_vec[k]], rows_v.at[t0 + k], sem).start()

    def drain_block():
        for _ in range(16):
            pltpu.make_async_copy(emb_hbm.at[0], rows_v.at[0], sem).wait()

    def pipe(i, _):
        fire_block(i)
        drain_block()
        return 0

    fire_block(0)
    lax.fori_loop(1, NBLK, pipe, 0)
    drain_block()

    for c in sgn_copies:
        c.wait()

    def tanh_chunk(i, _):
        x = sgn_v[pl.ds(i * 16, 16)]
        e = jnp.exp(-2.0 * jnp.abs(x))
        sgn_v[pl.ds(i * 16, 16)] = jnp.sign(x) * (1.0 - e) / (1.0 + e)
        return 0

    lax.fori_loop(0, NBLK, tanh_chunk, 0)

    def blk(i, carry):
        a0, a1 = carry
        s_vec = sgn_v[pl.ds(i * 16, 16)]
        t0 = i * 16
        for k in range(16):
            s = s_vec[k]
            a0 = a0 + s * rows_v[t0 + k, pl.ds(0, 16)]
            a1 = a1 + s * rows_v[t0 + k, pl.ds(16, 16)]
        return (a0, a1)

    z = jnp.zeros((16,), jnp.float32)
    a0, a1 = lax.fori_loop(0, NBLK, blk, (z, z))
    part_v[pl.ds(0, 16)] = a0
    part_v[pl.ds(16, 16)] = a1
    pltpu.sync_copy(part_v, out_hbm.at[wid])


def _reduce_body(p_ref, o_ref):
    o_ref[...] = jnp.sum(p_ref[...], axis=0, keepdims=True)


@jax.jit
def kernel(sequence, hash_embedding, sign_weight):
    seq = sequence.astype(jnp.int32)
    sc = pl.kernel(
        _sc_body,
        out_type=jax.ShapeDtypeStruct((NW, DIM), jnp.float32),
        mesh=plsc.VectorSubcoreMesh(core_axis_name="c", subcore_axis_name="s"),
        scratch_types=[
            pltpu.VMEM((NCHUNK, CHUNK), jnp.int32),
            pltpu.VMEM((TPW, DIM), jnp.float32),
            pltpu.VMEM((TPW,), jnp.float32),
            pltpu.VMEM((DIM,), jnp.float32),
            pltpu.SemaphoreType.DMA,
            pltpu.SemaphoreType.DMA,
        ],
    )
    partials = sc(seq, hash_embedding, sign_weight)
    out = pl.pallas_call(
        _reduce_body,
        out_shape=jax.ShapeDtypeStruct((1, DIM), jnp.float32),
    )(partials)
    return out.reshape(DIM)
